# baseline (speedup 1.0000x reference)
import jax
import jax.numpy as jnp
from jax.experimental import pallas as pl
from jax.experimental.pallas import tpu as pltpu


def _add_kernel(x_hbm, p_hbm, o_hbm, xbuf, obuf, pbuf, xsem, psem, osem):
    nb = x_hbm.shape[0]

    pltpu.make_async_copy(p_hbm, pbuf, psem).start()
    for i in range(nb):
        pltpu.make_async_copy(x_hbm.at[i], xbuf.at[i], xsem.at[i]).start()
    pltpu.make_async_copy(p_hbm, pbuf, psem).wait()

    for i in range(nb):
        pltpu.make_async_copy(x_hbm.at[i], xbuf.at[i], xsem.at[i]).wait()
        obuf[i] = xbuf[i] + pbuf[...]
        pltpu.make_async_copy(obuf.at[i], o_hbm.at[i], osem.at[i]).start()

    for i in range(nb):
        pltpu.make_async_copy(obuf.at[i], o_hbm.at[i], osem.at[i]).wait()


def kernel(x, pos_table):
    B, T, D = x.shape
    return pl.pallas_call(
        _add_kernel,
        in_specs=[
            pl.BlockSpec(memory_space=pl.ANY),
            pl.BlockSpec(memory_space=pl.ANY),
        ],
        out_specs=pl.BlockSpec(memory_space=pl.ANY),
        out_shape=jax.ShapeDtypeStruct((B, T, D), x.dtype),
        scratch_shapes=[
            pltpu.VMEM((B, T, D), x.dtype),
            pltpu.VMEM((B, T, D), x.dtype),
            pltpu.VMEM((T, D), x.dtype),
            pltpu.SemaphoreType.DMA((B,)),
            pltpu.SemaphoreType.DMA,
            pltpu.SemaphoreType.DMA((B,)),
        ],
    )(x, pos_table)


# speedup vs baseline: 1.0109x; 1.0109x over previous
import jax
import jax.numpy as jnp
from jax.experimental import pallas as pl
from jax.experimental.pallas import tpu as pltpu

_CHUNKS = [(0, 1), (1, 1), (2, 2)]


def _add_kernel(x_hbm, p_hbm, o_hbm, xbuf, obuf, pbuf, xsem, psem, osem):
    pltpu.make_async_copy(p_hbm, pbuf, psem).start()
    for i, (s, c) in enumerate(_CHUNKS):
        pltpu.make_async_copy(
            x_hbm.at[pl.ds(s, c)], xbuf.at[pl.ds(s, c)], xsem.at[i]
        ).start()
    pltpu.make_async_copy(p_hbm, pbuf, psem).wait()

    for i, (s, c) in enumerate(_CHUNKS):
        pltpu.make_async_copy(
            x_hbm.at[pl.ds(s, c)], xbuf.at[pl.ds(s, c)], xsem.at[i]
        ).wait()
        for b in range(s, s + c):
            obuf[b] = xbuf[b] + pbuf[...]
        pltpu.make_async_copy(
            obuf.at[pl.ds(s, c)], o_hbm.at[pl.ds(s, c)], osem.at[i]
        ).start()

    for i, (s, c) in enumerate(_CHUNKS):
        pltpu.make_async_copy(
            obuf.at[pl.ds(s, c)], o_hbm.at[pl.ds(s, c)], osem.at[i]
        ).wait()


def kernel(x, pos_table):
    B, T, D = x.shape
    n = len(_CHUNKS)
    return pl.pallas_call(
        _add_kernel,
        in_specs=[
            pl.BlockSpec(memory_space=pl.ANY),
            pl.BlockSpec(memory_space=pl.ANY),
        ],
        out_specs=pl.BlockSpec(memory_space=pl.ANY),
        out_shape=jax.ShapeDtypeStruct((B, T, D), x.dtype),
        scratch_shapes=[
            pltpu.VMEM((B, T, D), x.dtype),
            pltpu.VMEM((B, T, D), x.dtype),
            pltpu.VMEM((T, D), x.dtype),
            pltpu.SemaphoreType.DMA((n,)),
            pltpu.SemaphoreType.DMA,
            pltpu.SemaphoreType.DMA((n,)),
        ],
    )(x, pos_table)


# speedup vs baseline: 1.0132x; 1.0022x over previous
import jax
import jax.numpy as jnp
from jax.experimental import pallas as pl
from jax.experimental.pallas import tpu as pltpu


def _add_kernel(x_hbm, p_hbm, o_hbm, xbuf, pbuf, xsem, psem, osem):
    nb = x_hbm.shape[0] // 2

    pltpu.make_async_copy(p_hbm, pbuf, psem).start()
    for i in range(nb):
        pltpu.make_async_copy(
            x_hbm.at[pl.ds(2 * i, 2)], xbuf.at[i], xsem.at[i]
        ).start()
    pltpu.make_async_copy(p_hbm, pbuf, psem).wait()

    for i in range(nb):
        pltpu.make_async_copy(
            x_hbm.at[pl.ds(2 * i, 2)], xbuf.at[i], xsem.at[i]
        ).wait()
        xbuf[i] = xbuf[i] + pbuf[...]
        pltpu.make_async_copy(
            xbuf.at[i], o_hbm.at[pl.ds(2 * i, 2)], osem.at[i]
        ).start()

    for i in range(nb):
        pltpu.make_async_copy(
            xbuf.at[i], o_hbm.at[pl.ds(2 * i, 2)], osem.at[i]
        ).wait()


def kernel(x, pos_table):
    B, T, D = x.shape
    return pl.pallas_call(
        _add_kernel,
        in_specs=[
            pl.BlockSpec(memory_space=pl.ANY),
            pl.BlockSpec(memory_space=pl.ANY),
        ],
        out_specs=pl.BlockSpec(memory_space=pl.ANY),
        out_shape=jax.ShapeDtypeStruct((B, T, D), x.dtype),
        scratch_shapes=[
            pltpu.VMEM((B // 2, 2, T, D), x.dtype),
            pltpu.VMEM((T, D), x.dtype),
            pltpu.SemaphoreType.DMA((B // 2,)),
            pltpu.SemaphoreType.DMA,
            pltpu.SemaphoreType.DMA((B // 2,)),
        ],
    )(x, pos_table)
